# baseline (speedup 1.0000x reference)
import functools

import jax
import jax.numpy as jnp
import numpy as np
from jax import lax
from jax.experimental import pallas as pl
from jax.experimental.pallas import tpu as pltpu
from jax.experimental.pallas import tpu_sc as plsc

N = 50000
H = 16
KP = 15
CIN = 64
COUT = 64
KP_EXTENT = 1.2

T = 128
NPAD = 50176
B = NPAD * H
BH = B // 2
NH = NPAD // 2
NW = 32
PER_W = BH // NW
CH = 784
NCHUNK = PER_W // CH


_SEL = np.zeros((8, 128, 1024), np.float32)
for _j in range(8):
    for _u in range(2):
        for _a in range(8):
            _SEL[_j, 16 * _a + (2 * _j + _u),
                 512 * _u + 64 * _a : 512 * _u + 64 * _a + 64] = 1.0

_R = np.zeros((256, 128), np.float32)
for _r in range(256):
    _R[_r, _r // 2] = 1.0
_T8 = np.zeros((16, 128), np.float32)
for _g in range(8):
    for _j in range(16):
        _T8[_j, 16 * _g + _j] = 1.0


def _blockdiag(block, n):
    m = np.zeros((block.shape[0] * n, block.shape[1] * n), np.float32)
    for i in range(n):
        m[i * block.shape[0] : (i + 1) * block.shape[0],
          i * block.shape[1] : (i + 1) * block.shape[1]] = block
    return m


_ONESBD16 = _blockdiag(np.ones((16, 16), np.float32), 8)


def _sc_gather_body(xpad, spad, idx, outf, outc,
                    idx0, idx1, rf0, rf1, rc0, rc1,
                    gf0, gf1, gc0, gc1, sf0, sf1, sc0, sc1):
    wid = lax.axis_index("s") * 2 + lax.axis_index("c")
    wbase = wid * PER_W

    idx_v = (idx0, idx1)
    rf = (rf0, rf1)
    rc = (rc0, rc1)
    gfs = (gf0, gf1)
    gcs = (gc0, gc1)
    sfs = (sf0, sf1)
    scs = (sc0, sc1)

    def start_gather(j, b):
        pltpu.sync_copy(idx.at[pl.ds(wbase + j * CH, CH)], idx_v[b])
        pltpu.async_copy(xpad.at[idx_v[b]], rf[b], gfs[b])
        pltpu.async_copy(spad.at[idx_v[b]], rc[b], gcs[b])

    def wait_gather(b):
        pltpu.make_async_copy(xpad.at[idx_v[b]], rf[b], gfs[b]).wait()
        pltpu.make_async_copy(spad.at[idx_v[b]], rc[b], gcs[b]).wait()

    def start_store(j, b):
        pltpu.async_copy(rf[b], outf.at[pl.ds(wbase + j * CH, CH)], sfs[b])
        pltpu.async_copy(rc[b], outc.at[pl.ds(wbase + j * CH, CH)], scs[b])

    def wait_store(b):
        pltpu.make_async_copy(rf[b], outf.at[pl.ds(0, CH)], sfs[b]).wait()
        pltpu.make_async_copy(rc[b], outc.at[pl.ds(0, CH)], scs[b]).wait()

    start_gather(0, 0)

    def body(jj, carry):
        a = 2 * jj + 1
        b2 = 2 * jj + 2

        @pl.when(jj >= 1)
        def _():
            wait_store(1)

        start_gather(a, 1)
        wait_gather(0)
        start_store(2 * jj, 0)
        wait_store(0)
        start_gather(b2, 0)
        wait_gather(1)
        start_store(a, 1)
        return carry

    lax.fori_loop(0, (NCHUNK - 2) // 2, body, 0)

    wait_store(1)
    start_gather(NCHUNK - 1, 1)
    wait_gather(0)
    start_store(NCHUNK - 2, 0)
    wait_gather(1)
    start_store(NCHUNK - 1, 1)
    wait_store(0)
    wait_store(1)


@functools.cache
def _sc_gather():
    return pl.kernel(
        _sc_gather_body,
        out_type=[
            jax.ShapeDtypeStruct((BH, CIN), jnp.float32),
            jax.ShapeDtypeStruct((BH, 16), jnp.float32),
        ],
        mesh=plsc.VectorSubcoreMesh(
            core_axis_name="c", subcore_axis_name="s", num_cores=2, num_subcores=16
        ),
        scratch_types=[
            pltpu.VMEM((CH,), jnp.int32),
            pltpu.VMEM((CH,), jnp.int32),
            pltpu.VMEM((CH, CIN), jnp.float32),
            pltpu.VMEM((CH, CIN), jnp.float32),
            pltpu.VMEM((CH, 16), jnp.float32),
            pltpu.VMEM((CH, 16), jnp.float32),
        ] + [pltpu.SemaphoreType.DMA] * 8,
        compiler_params=pltpu.CompilerParams(use_tc_tiling_on_sc=False),
    )


def _tc_body(q_ref, gf_ref, gc_ref, kptbd_ref, obd16_ref,
             r_ref, t8_ref, sel_ref, ws2_ref, o_ref):
    HI = jax.lax.Precision.HIGHEST
    f32 = jnp.float32
    RQ = T * H // 8
    F8 = gf_ref[:]
    Cp = gc_ref[:]
    q = q_ref[:]

    qw = jnp.dot(q, t8_ref[:], preferred_element_type=f32, precision=HI)
    qrep = jnp.dot(r_ref[:], qw, preferred_element_type=f32, precision=HI)
    dp = Cp - qrep
    sb = jnp.dot(dp * dp, obd16_ref[:], preferred_element_type=f32,
                 precision=HI)
    kptbd = kptbd_ref[:]
    dk = jnp.dot(dp, kptbd, preferred_element_type=f32, precision=HI)
    kpn2 = jnp.sum(kptbd * kptbd, axis=0, keepdims=True)
    d2 = jnp.maximum(sb - 2.0 * dk + kpn2, 0.0)
    Ap = jnp.maximum(1.0 - jnp.sqrt(d2) * (1.0 / KP_EXTENT), 0.0)
    Apb = Ap.astype(jnp.bfloat16)

    F8b = F8.astype(jnp.bfloat16)
    Fd = jnp.concatenate([F8b, F8b], axis=1)
    S = jnp.zeros((RQ, COUT), dtype=f32)
    for j in range(8):
        A4 = jnp.dot(Apb, sel_ref[j],
                     preferred_element_type=f32).astype(jnp.bfloat16)
        S = S + jnp.dot(A4 * Fd, ws2_ref[j], preferred_element_type=f32)
    outp = S.reshape(T, 2, COUT).sum(axis=1)

    cnt1 = jnp.zeros((RQ, 1), dtype=f32)
    for a in range(8):
        fs = jnp.sum(F8[:, 64 * a : 64 * (a + 1)], axis=1, keepdims=True)
        cnt1 = cnt1 + (fs > 0.0).astype(f32)
    cntp = cnt1.reshape(T, 2, 1).sum(axis=1)
    o_ref[:] = outp / jnp.maximum(cntp, 1.0)


def _tc_call(qp, gf2, gc8, kptbd, obd16, rmat, t8, sel, ws2):
    grid = (NH // T,)
    return pl.pallas_call(
        _tc_body,
        grid=grid,
        in_specs=[
            pl.BlockSpec((T, 16), lambda i: (i, 0)),
            pl.BlockSpec((T * H // 8, 8 * CIN), lambda i: (i, 0)),
            pl.BlockSpec((T * H // 8, 128), lambda i: (i, 0)),
            pl.BlockSpec((128, 128), lambda i: (0, 0)),
            pl.BlockSpec((128, 128), lambda i: (0, 0)),
            pl.BlockSpec((256, 128), lambda i: (0, 0)),
            pl.BlockSpec((16, 128), lambda i: (0, 0)),
            pl.BlockSpec((8, 128, 1024), lambda i: (0, 0, 0)),
            pl.BlockSpec((8, 1024, COUT), lambda i: (0, 0, 0)),
        ],
        out_specs=pl.BlockSpec((T, COUT), lambda i: (i, 0)),
        out_shape=jax.ShapeDtypeStruct((NH, COUT), jnp.float32),
    )(qp, gf2, gc8, kptbd, obd16, rmat, t8, sel, ws2)


@jax.jit
def kernel(q_pts, s_pts, neighb_inds, x, kernel_points, weights):
    xpad = jnp.concatenate([x, jnp.zeros((1, CIN), jnp.float32)], axis=0)
    spad = jnp.zeros((N + 1, 16), jnp.float32)
    spad = spad.at[:N, :3].set(s_pts)
    spad = spad.at[N].set(1e6)
    idx = jnp.full((NPAD, H), N, jnp.int32)
    idx = idx.at[:N].set(neighb_inds).reshape(-1)
    qp = jnp.zeros((NPAD, 16), jnp.float32).at[:N, :3].set(q_pts)
    kpt16 = jnp.zeros((16, 16), jnp.float32).at[:3, :15].set(kernel_points.T)
    kptbd = jax.scipy.linalg.block_diag(*([kpt16] * 8))
    sel = jnp.asarray(_SEL).astype(jnp.bfloat16)
    wpad = jnp.concatenate(
        [weights, jnp.zeros((1, CIN, COUT), jnp.float32)], axis=0
    )
    ws2 = jnp.repeat(
        wpad.reshape(8, 2, 1, CIN, COUT), 8, axis=2
    ).reshape(8, 16 * CIN, COUT).astype(jnp.bfloat16)

    consts = (kptbd, jnp.asarray(_ONESBD16), jnp.asarray(_R),
              jnp.asarray(_T8), sel, ws2)
    outs = []
    for hfi in range(2):
        idx_h = lax.dynamic_slice_in_dim(idx, hfi * BH, BH)
        qp_h = lax.dynamic_slice_in_dim(qp, hfi * NH, NH)
        gf, gc = _sc_gather()(xpad, spad, idx_h)
        gf8 = jnp.reshape(gf, (BH // 8, 8 * CIN))
        gc8 = jnp.reshape(gc, (BH // 8, 128))
        outs.append(_tc_call(qp_h, gf8, gc8, *consts))
    out = jnp.concatenate(outs, axis=0)
    return out[:N]


# speedup vs baseline: 1.0027x; 1.0027x over previous
import functools

import jax
import jax.numpy as jnp
import numpy as np
from jax import lax
from jax.experimental import pallas as pl
from jax.experimental.pallas import tpu as pltpu
from jax.experimental.pallas import tpu_sc as plsc

N = 50000
H = 16
KP = 15
CIN = 64
COUT = 64
KP_EXTENT = 1.2

T = 128
NPAD = 50176
B = NPAD * H
BH = B // 2
NH = NPAD // 2
NW = 32
PER_W = BH // NW
CH = 448
NCHUNK = PER_W // CH


_SEL = np.zeros((8, 128, 1024), np.float32)
for _j in range(8):
    for _u in range(2):
        for _a in range(8):
            _SEL[_j, 16 * _a + (2 * _j + _u),
                 512 * _u + 64 * _a : 512 * _u + 64 * _a + 64] = 1.0

_R = np.zeros((256, 128), np.float32)
for _r in range(256):
    _R[_r, _r // 2] = 1.0
_T8 = np.zeros((16, 128), np.float32)
for _g in range(8):
    for _j in range(16):
        _T8[_j, 16 * _g + _j] = 1.0


def _blockdiag(block, n):
    m = np.zeros((block.shape[0] * n, block.shape[1] * n), np.float32)
    for i in range(n):
        m[i * block.shape[0] : (i + 1) * block.shape[0],
          i * block.shape[1] : (i + 1) * block.shape[1]] = block
    return m


_ONESBD16 = _blockdiag(np.ones((16, 16), np.float32), 8)


def _sc_gather_body(xpad, spad, idx, outf, outc,
                    idx0, idx1, rf0, rf1, rc0, rc1,
                    gf0, gf1, gc0, gc1, sf0, sf1, sc0, sc1):
    wid = lax.axis_index("s") * 2 + lax.axis_index("c")
    wbase = wid * PER_W

    idx_v = (idx0, idx1)
    rf = (rf0, rf1)
    rc = (rc0, rc1)
    gfs = (gf0, gf1)
    gcs = (gc0, gc1)
    sfs = (sf0, sf1)
    scs = (sc0, sc1)

    def start_gather(j, b):
        pltpu.sync_copy(idx.at[pl.ds(wbase + j * CH, CH)], idx_v[b])
        pltpu.async_copy(xpad.at[idx_v[b]], rf[b], gfs[b])
        pltpu.async_copy(spad.at[idx_v[b]], rc[b], gcs[b])

    def wait_gather(b):
        pltpu.make_async_copy(xpad.at[idx_v[b]], rf[b], gfs[b]).wait()
        pltpu.make_async_copy(spad.at[idx_v[b]], rc[b], gcs[b]).wait()

    def start_store(j, b):
        pltpu.async_copy(rf[b], outf.at[pl.ds(wbase + j * CH, CH)], sfs[b])
        pltpu.async_copy(rc[b], outc.at[pl.ds(wbase + j * CH, CH)], scs[b])

    def wait_store(b):
        pltpu.make_async_copy(rf[b], outf.at[pl.ds(0, CH)], sfs[b]).wait()
        pltpu.make_async_copy(rc[b], outc.at[pl.ds(0, CH)], scs[b]).wait()

    start_gather(0, 0)

    def body(jj, carry):
        a = 2 * jj + 1
        b2 = 2 * jj + 2

        @pl.when(jj >= 1)
        def _():
            wait_store(1)

        start_gather(a, 1)
        wait_gather(0)
        start_store(2 * jj, 0)
        wait_store(0)
        start_gather(b2, 0)
        wait_gather(1)
        start_store(a, 1)
        return carry

    lax.fori_loop(0, (NCHUNK - 2) // 2, body, 0)

    wait_store(1)
    start_gather(NCHUNK - 1, 1)
    wait_gather(0)
    start_store(NCHUNK - 2, 0)
    wait_gather(1)
    start_store(NCHUNK - 1, 1)
    wait_store(0)
    wait_store(1)


@functools.cache
def _sc_gather():
    return pl.kernel(
        _sc_gather_body,
        out_type=[
            jax.ShapeDtypeStruct((BH, CIN), jnp.float32),
            jax.ShapeDtypeStruct((BH, 16), jnp.float32),
        ],
        mesh=plsc.VectorSubcoreMesh(
            core_axis_name="c", subcore_axis_name="s", num_cores=2, num_subcores=16
        ),
        scratch_types=[
            pltpu.VMEM((CH,), jnp.int32),
            pltpu.VMEM((CH,), jnp.int32),
            pltpu.VMEM((CH, CIN), jnp.float32),
            pltpu.VMEM((CH, CIN), jnp.float32),
            pltpu.VMEM((CH, 16), jnp.float32),
            pltpu.VMEM((CH, 16), jnp.float32),
        ] + [pltpu.SemaphoreType.DMA] * 8,
        compiler_params=pltpu.CompilerParams(use_tc_tiling_on_sc=False),
    )


def _tc_body(q_ref, gf_ref, gc_ref, kptbd_ref, obd16_ref,
             r_ref, t8_ref, sel_ref, ws2_ref, o_ref):
    HI = jax.lax.Precision.HIGHEST
    f32 = jnp.float32
    RQ = T * H // 8
    F8 = gf_ref[:]
    Cp = gc_ref[:]
    q = q_ref[:]

    qw = jnp.dot(q, t8_ref[:], preferred_element_type=f32, precision=HI)
    qrep = jnp.dot(r_ref[:], qw, preferred_element_type=f32, precision=HI)
    dp = Cp - qrep
    sb = jnp.dot(dp * dp, obd16_ref[:], preferred_element_type=f32,
                 precision=HI)
    kptbd = kptbd_ref[:]
    dk = jnp.dot(dp, kptbd, preferred_element_type=f32, precision=HI)
    kpn2 = jnp.sum(kptbd * kptbd, axis=0, keepdims=True)
    d2 = jnp.maximum(sb - 2.0 * dk + kpn2, 0.0)
    Ap = jnp.maximum(1.0 - jnp.sqrt(d2) * (1.0 / KP_EXTENT), 0.0)
    Apb = Ap.astype(jnp.bfloat16)

    F8b = F8.astype(jnp.bfloat16)
    Fd = jnp.concatenate([F8b, F8b], axis=1)
    S = jnp.zeros((RQ, COUT), dtype=f32)
    for j in range(8):
        A4 = jnp.dot(Apb, sel_ref[j],
                     preferred_element_type=f32).astype(jnp.bfloat16)
        S = S + jnp.dot(A4 * Fd, ws2_ref[j], preferred_element_type=f32)
    outp = S.reshape(T, 2, COUT).sum(axis=1)

    cnt1 = jnp.zeros((RQ, 1), dtype=f32)
    for a in range(8):
        fs = jnp.sum(F8[:, 64 * a : 64 * (a + 1)], axis=1, keepdims=True)
        cnt1 = cnt1 + (fs > 0.0).astype(f32)
    cntp = cnt1.reshape(T, 2, 1).sum(axis=1)
    o_ref[:] = outp / jnp.maximum(cntp, 1.0)


def _tc_call(qp, gf2, gc8, kptbd, obd16, rmat, t8, sel, ws2):
    grid = (NH // T,)
    return pl.pallas_call(
        _tc_body,
        grid=grid,
        in_specs=[
            pl.BlockSpec((T, 16), lambda i: (i, 0)),
            pl.BlockSpec((T * H // 8, 8 * CIN), lambda i: (i, 0)),
            pl.BlockSpec((T * H // 8, 128), lambda i: (i, 0)),
            pl.BlockSpec((128, 128), lambda i: (0, 0)),
            pl.BlockSpec((128, 128), lambda i: (0, 0)),
            pl.BlockSpec((256, 128), lambda i: (0, 0)),
            pl.BlockSpec((16, 128), lambda i: (0, 0)),
            pl.BlockSpec((8, 128, 1024), lambda i: (0, 0, 0)),
            pl.BlockSpec((8, 1024, COUT), lambda i: (0, 0, 0)),
        ],
        out_specs=pl.BlockSpec((T, COUT), lambda i: (i, 0)),
        out_shape=jax.ShapeDtypeStruct((NH, COUT), jnp.float32),
    )(qp, gf2, gc8, kptbd, obd16, rmat, t8, sel, ws2)


@jax.jit
def kernel(q_pts, s_pts, neighb_inds, x, kernel_points, weights):
    xpad = jnp.concatenate([x, jnp.zeros((1, CIN), jnp.float32)], axis=0)
    spad = jnp.zeros((N + 1, 16), jnp.float32)
    spad = spad.at[:N, :3].set(s_pts)
    spad = spad.at[N].set(1e6)
    idx = jnp.full((NPAD, H), N, jnp.int32)
    idx = idx.at[:N].set(neighb_inds).reshape(-1)
    qp = jnp.zeros((NPAD, 16), jnp.float32).at[:N, :3].set(q_pts)
    kpt16 = jnp.zeros((16, 16), jnp.float32).at[:3, :15].set(kernel_points.T)
    kptbd = jax.scipy.linalg.block_diag(*([kpt16] * 8))
    sel = jnp.asarray(_SEL).astype(jnp.bfloat16)
    wpad = jnp.concatenate(
        [weights, jnp.zeros((1, CIN, COUT), jnp.float32)], axis=0
    )
    ws2 = jnp.repeat(
        wpad.reshape(8, 2, 1, CIN, COUT), 8, axis=2
    ).reshape(8, 16 * CIN, COUT).astype(jnp.bfloat16)

    consts = (kptbd, jnp.asarray(_ONESBD16), jnp.asarray(_R),
              jnp.asarray(_T8), sel, ws2)
    outs = []
    for hfi in range(2):
        idx_h = lax.dynamic_slice_in_dim(idx, hfi * BH, BH)
        qp_h = lax.dynamic_slice_in_dim(qp, hfi * NH, NH)
        gf, gc = _sc_gather()(xpad, spad, idx_h)
        gf8 = jnp.reshape(gf, (BH // 8, 8 * CIN))
        gc8 = jnp.reshape(gc, (BH // 8, 128))
        outs.append(_tc_call(qp_h, gf8, gc8, *consts))
    out = jnp.concatenate(outs, axis=0)
    return out[:N]
